# baseline (speedup 1.0000x reference)
import jax
import jax.numpy as jnp
from jax.experimental import pallas as pl
from jax.experimental.pallas import tpu as pltpu

IN_DIM = 768
E_DIM = 64
NUM_LEVELS = 3
K = 1024
BETA = 0.25
BATCH = 16384
TILE = 512
NSTREAM = 4
BLOCK = TILE * NSTREAM
_CHUNK = 128

_DN = lambda lc, rc: ((lc, rc), ((), ()))


def _dot(a, b, dims=(((1,), (0,)), ((), ()))):
    return jax.lax.dot_general(a, b, dims,
                               precision=jax.lax.Precision.DEFAULT,
                               preferred_element_type=jnp.float32)


def _argmin_sublanes(d):
    vals = [d[8 * h:8 * (h + 1)] for h in range(K // 8)]
    idxs = None
    while len(vals) > 1:
        if idxs is None:
            nv, ni = [], []
            for j in range(0, len(vals), 2):
                a, b = vals[j], vals[j + 1]
                mask = b < a
                nv.append(jnp.where(mask, b, a))
                ni.append(jnp.where(mask, jnp.int32(j + 1), jnp.int32(j)))
            vals, idxs = nv, ni
        else:
            nv, ni = [], []
            for j in range(0, len(vals), 2):
                mask = vals[j + 1] < vals[j]
                nv.append(jnp.where(mask, vals[j + 1], vals[j]))
                ni.append(jnp.where(mask, idxs[j + 1], idxs[j]))
            vals, idxs = nv, ni
    val, idx = vals[0], idxs[0]
    r = idx * 8 + jax.lax.broadcasted_iota(jnp.int32, val.shape, 0)
    m8 = jnp.min(val, axis=0, keepdims=True)
    return jnp.min(jnp.where(val == m8, r, K), axis=0)


def _gather_rows(cbT, idx):
    lo = jnp.bitwise_and(idx, _CHUNK - 1)
    hi = jnp.right_shift(idx, 7)
    lo_b = jax.lax.broadcast_in_dim(lo, (E_DIM, TILE), (1,))
    hi_b = jax.lax.broadcast_in_dim(hi, (E_DIM, TILE), (1,))
    xqT = jnp.zeros((E_DIM, TILE), jnp.float32)
    for h in range(K // _CHUNK):
        g = jnp.take_along_axis(cbT[:, h * _CHUNK:(h + 1) * _CHUNK], lo_b,
                                axis=1)
        xqT = jnp.where(hi_b == h, g, xqT)
    return xqT


def _rqvae_kernel(x_ref, ew0, eb0, ew1, eb1, ew2, eb2, cbT_ref, cbTm2_ref,
                  dw0, db0, dw1, db1, dw2, db2,
                  out_ref, idx_ref, loss_ref, res_s):
    i = pl.program_id(0)
    resT_all = res_s[...]
    rs = [resT_all[:, s * TILE:(s + 1) * TILE] for s in range(NSTREAM)]

    accs = [jnp.zeros((E_DIM, TILE), jnp.float32) for _ in range(NSTREAM)]
    c2s = [jnp.sum(cbT_ref[l] * cbT_ref[l], axis=0)[:, None]
           for l in range(NUM_LEVELS)]
    loss_sums = []
    idx_rows = [[] for _ in range(NSTREAM)]
    for lvl in range(NUM_LEVELS):
        cbT = cbT_ref[lvl]
        ds = [c2s[lvl] + _dot(cbTm2_ref[lvl], r, _DN((0,), (0,)))
              for r in rs]
        idxs = [_argmin_sublanes(d) for d in ds]
        xqs = [_gather_rows(cbT, idx) for idx in idxs]
        diffs = [xq - r for xq, r in zip(xqs, rs)]
        loss_sums.append(sum(jnp.sum(df * df) for df in diffs))
        accs = [a + xq for a, xq in zip(accs, xqs)]
        rs = [r - xq for r, xq in zip(rs, xqs)]
        for s in range(NSTREAM):
            idx_rows[s].append(idxs[s])

    acc_all = jnp.concatenate(accs, axis=1)
    h = jnp.maximum(_dot(acc_all, dw0[...], _DN((0,), (0,))) + db0[...], 0.0)
    h = jnp.maximum(_dot(h, dw1[...]) + db1[...], 0.0)
    out_ref[...] = jax.nn.sigmoid(_dot(h, dw2[...]) + db2[...])
    for s in range(NSTREAM):
        idx_ref[:, s * TILE:(s + 1) * TILE] = jnp.stack(idx_rows[s], axis=0)

    cur = jnp.stack(loss_sums)[None, :]
    prev = jnp.where(i <= 1, jnp.zeros_like(cur), loss_ref[...])
    loss_ref[...] = prev + jnp.where(i >= 1, cur, jnp.zeros_like(cur))

    h = jnp.maximum(_dot(x_ref[...], ew0[...]) + eb0[...], 0.0)
    h = jnp.maximum(_dot(h, ew1[...]) + eb1[...], 0.0)
    res_s[...] = _dot(ew2[...], h, _DN((0,), (1,))) + eb2[...]


@jax.jit
def _run(x, enc_W0, enc_b0, enc_W1, enc_b1, enc_W2, enc_b2,
         codebooks, dec_W0, dec_b0, dec_W1, dec_b1, dec_W2, dec_b2):
    nblk = BATCH // BLOCK
    full = lambda shape: pl.BlockSpec(shape, lambda i: (0,) * len(shape))
    cbT = codebooks.transpose(0, 2, 1)
    cbTm2 = -2.0 * cbT
    out, idxs, loss = pl.pallas_call(
        _rqvae_kernel,
        grid=(nblk + 1,),
        scratch_shapes=[pltpu.VMEM((E_DIM, BLOCK), jnp.float32)],
        in_specs=[
            pl.BlockSpec((BLOCK, IN_DIM),
                         lambda i: (jnp.minimum(i, nblk - 1), 0)),
            full(enc_W0.shape), full((1, enc_b0.shape[0])),
            full(enc_W1.shape), full((1, enc_b1.shape[0])),
            full(enc_W2.shape), full((enc_b2.shape[0], 1)),
            full(cbT.shape), full(cbTm2.shape),
            full(dec_W0.shape), full((1, dec_b0.shape[0])),
            full(dec_W1.shape), full((1, dec_b1.shape[0])),
            full(dec_W2.shape), full((1, dec_b2.shape[0])),
        ],
        out_specs=[
            pl.BlockSpec((BLOCK, IN_DIM),
                         lambda i: (jnp.maximum(i - 1, 0), 0)),
            pl.BlockSpec((NUM_LEVELS, BLOCK),
                         lambda i: (0, jnp.maximum(i - 1, 0))),
            pl.BlockSpec((1, NUM_LEVELS), lambda i: (0, 0)),
        ],
        out_shape=[
            jax.ShapeDtypeStruct((BATCH, IN_DIM), jnp.float32),
            jax.ShapeDtypeStruct((NUM_LEVELS, BATCH), jnp.int32),
            jax.ShapeDtypeStruct((1, NUM_LEVELS), jnp.float32),
        ],
    )(x, enc_W0, enc_b0.reshape(1, -1), enc_W1, enc_b1.reshape(1, -1),
      enc_W2, enc_b2.reshape(-1, 1), cbT, cbTm2,
      dec_W0, dec_b0.reshape(1, -1), dec_W1, dec_b1.reshape(1, -1),
      dec_W2, dec_b2.reshape(1, -1))
    per_level_mse = loss[0] / (BATCH * E_DIM)
    rq_loss = jnp.mean((1.0 + BETA) * per_level_mse)
    return out, rq_loss, idxs.T


def kernel(x, epoch_idx, enc_W0, enc_b0, enc_W1, enc_b1, enc_W2, enc_b2,
           codebooks, dec_W0, dec_b0, dec_W1, dec_b1, dec_W2, dec_b2):
    return _run(x, enc_W0, enc_b0, enc_W1, enc_b1, enc_W2, enc_b2,
                codebooks, dec_W0, dec_b0, dec_W1, dec_b1, dec_W2, dec_b2)


# speedup vs baseline: 1.2785x; 1.2785x over previous
import jax
import jax.numpy as jnp
from jax.experimental import pallas as pl

IN_DIM = 768
E_DIM = 64
NUM_LEVELS = 3
K = 1024
BETA = 0.25
BATCH = 16384
TILE = 512
NSTREAM = 4
BLOCK = TILE * NSTREAM
_CHUNK = 128

_DN = lambda lc, rc: ((lc, rc), ((), ()))


def _dot(a, b, dims=(((1,), (0,)), ((), ()))):
    return jax.lax.dot_general(a, b, dims,
                               precision=jax.lax.Precision.DEFAULT,
                               preferred_element_type=jnp.float32)


def _argmin_sublanes(d):
    vals = [d[8 * h:8 * (h + 1)] for h in range(K // 8)]
    idxs = None
    while len(vals) > 1:
        if idxs is None:
            nv, ni = [], []
            for j in range(0, len(vals), 2):
                a, b = vals[j], vals[j + 1]
                mask = b < a
                nv.append(jnp.where(mask, b, a))
                ni.append(jnp.where(mask, jnp.int32(j + 1), jnp.int32(j)))
            vals, idxs = nv, ni
        else:
            nv, ni = [], []
            for j in range(0, len(vals), 2):
                mask = vals[j + 1] < vals[j]
                nv.append(jnp.where(mask, vals[j + 1], vals[j]))
                ni.append(jnp.where(mask, idxs[j + 1], idxs[j]))
            vals, idxs = nv, ni
    val, idx = vals[0], idxs[0]
    r = idx * 8 + jax.lax.broadcasted_iota(jnp.int32, val.shape, 0)
    m8 = jnp.min(val, axis=0, keepdims=True)
    return jnp.min(jnp.where(val == m8, r, K), axis=0)


def _gather_rows(cbT, idx):
    lo = jnp.bitwise_and(idx, _CHUNK - 1)
    hi = jnp.right_shift(idx, 7)
    lo_b = jax.lax.broadcast_in_dim(lo, (E_DIM, TILE), (1,))
    hi_b = jax.lax.broadcast_in_dim(hi, (E_DIM, TILE), (1,))
    xqT = jnp.zeros((E_DIM, TILE), jnp.float32)
    for h in range(K // _CHUNK):
        g = jnp.take_along_axis(cbT[:, h * _CHUNK:(h + 1) * _CHUNK], lo_b,
                                axis=1)
        xqT = jnp.where(hi_b == h, g, xqT)
    return xqT


def _rqvae_kernel(x_ref, ew0, eb0, ew1, eb1, ew2, eb2, cbT_ref, cbTm2_ref,
                  dw0, db0, dw1, db1, dw2, db2,
                  out_ref, idx_ref, loss_ref):
    i = pl.program_id(0)
    h = jnp.maximum(_dot(x_ref[...], ew0[...]) + eb0[...], 0.0)
    h = jnp.maximum(_dot(h, ew1[...]) + eb1[...], 0.0)
    resT_all = _dot(ew2[...], h, _DN((0,), (1,))) + eb2[...]
    rs = [resT_all[:, s * TILE:(s + 1) * TILE] for s in range(NSTREAM)]

    accs = [jnp.zeros((E_DIM, TILE), jnp.float32) for _ in range(NSTREAM)]
    c2s = [jnp.sum(cbT_ref[l] * cbT_ref[l], axis=0)[:, None]
           for l in range(NUM_LEVELS)]
    loss_sums = []
    idx_rows = [[] for _ in range(NSTREAM)]
    for lvl in range(NUM_LEVELS):
        cbT = cbT_ref[lvl]
        ds = [c2s[lvl] + _dot(cbTm2_ref[lvl], r, _DN((0,), (0,)))
              for r in rs]
        idxs = [_argmin_sublanes(d) for d in ds]
        xqs = [_gather_rows(cbT, idx) for idx in idxs]
        diffs = [xq - r for xq, r in zip(xqs, rs)]
        loss_sums.append(sum(jnp.sum(df * df) for df in diffs))
        accs = [a + xq for a, xq in zip(accs, xqs)]
        rs = [r - xq for r, xq in zip(rs, xqs)]
        for s in range(NSTREAM):
            idx_rows[s].append(idxs[s])

    acc_all = jnp.concatenate(accs, axis=1)
    h = jnp.maximum(_dot(acc_all, dw0[...], _DN((0,), (0,))) + db0[...], 0.0)
    h = jnp.maximum(_dot(h, dw1[...]) + db1[...], 0.0)
    out_ref[...] = jax.nn.sigmoid(_dot(h, dw2[...]) + db2[...])
    for s in range(NSTREAM):
        idx_ref[:, s * TILE:(s + 1) * TILE] = jnp.stack(idx_rows[s], axis=0)

    @pl.when(i == 0)
    def _():
        loss_ref[...] = jnp.zeros_like(loss_ref)
    loss_ref[...] += jnp.stack(loss_sums)[None, :]


@jax.jit
def _run(x, enc_W0, enc_b0, enc_W1, enc_b1, enc_W2, enc_b2,
         codebooks, dec_W0, dec_b0, dec_W1, dec_b1, dec_W2, dec_b2):
    grid = BATCH // BLOCK
    full = lambda shape: pl.BlockSpec(shape, lambda i: (0,) * len(shape))
    cbT = codebooks.transpose(0, 2, 1)
    cbTm2 = -2.0 * cbT
    out, idxs, loss = pl.pallas_call(
        _rqvae_kernel,
        grid=(grid,),
        in_specs=[
            pl.BlockSpec((BLOCK, IN_DIM), lambda i: (i, 0)),
            full(enc_W0.shape), full((1, enc_b0.shape[0])),
            full(enc_W1.shape), full((1, enc_b1.shape[0])),
            full(enc_W2.shape), full((enc_b2.shape[0], 1)),
            full(cbT.shape), full(cbTm2.shape),
            full(dec_W0.shape), full((1, dec_b0.shape[0])),
            full(dec_W1.shape), full((1, dec_b1.shape[0])),
            full(dec_W2.shape), full((1, dec_b2.shape[0])),
        ],
        out_specs=[
            pl.BlockSpec((BLOCK, IN_DIM), lambda i: (i, 0)),
            pl.BlockSpec((NUM_LEVELS, BLOCK), lambda i: (0, i)),
            pl.BlockSpec((1, NUM_LEVELS), lambda i: (0, 0)),
        ],
        out_shape=[
            jax.ShapeDtypeStruct((BATCH, IN_DIM), jnp.float32),
            jax.ShapeDtypeStruct((NUM_LEVELS, BATCH), jnp.int32),
            jax.ShapeDtypeStruct((1, NUM_LEVELS), jnp.float32),
        ],
    )(x, enc_W0, enc_b0.reshape(1, -1), enc_W1, enc_b1.reshape(1, -1),
      enc_W2, enc_b2.reshape(-1, 1), cbT, cbTm2,
      dec_W0, dec_b0.reshape(1, -1), dec_W1, dec_b1.reshape(1, -1),
      dec_W2, dec_b2.reshape(1, -1))
    per_level_mse = loss[0] / (BATCH * E_DIM)
    rq_loss = jnp.mean((1.0 + BETA) * per_level_mse)
    return out, rq_loss, idxs.T


def kernel(x, epoch_idx, enc_W0, enc_b0, enc_W1, enc_b1, enc_W2, enc_b2,
           codebooks, dec_W0, dec_b0, dec_W1, dec_b1, dec_W2, dec_b2):
    return _run(x, enc_W0, enc_b0, enc_W1, enc_b1, enc_W2, enc_b2,
                codebooks, dec_W0, dec_b0, dec_W1, dec_b1, dec_W2, dec_b2)
